# baseline (speedup 1.0000x reference)
import functools

import jax
import jax.numpy as jnp
from jax import lax
from jax.experimental import pallas as pl
from jax.experimental.pallas import tpu as pltpu
from jax.experimental.pallas import tpu_sc as plsc

B = 128
NI = 8192
NN = 16384
FI = 16
NC = 2
NS = 16
NW = NC * NS
NPT = NN // NW
CH = 4
NCH = NPT // CH
ROWS = CH * FI
GRP = 4 * CH
NGRP = NPT // GRP
LANES = 16
NV = B // LANES
NU = B // (2 * LANES)


def _prep_e_body(x_ref, e_ref):
    e_ref[...] = jnp.transpose(jnp.exp(x_ref[...])).astype(jnp.bfloat16)


def _prep_e(x):
    return pl.pallas_call(
        _prep_e_body,
        grid=(16,),
        in_specs=[pl.BlockSpec((B, NI // 16), lambda i: (0, i))],
        out_specs=pl.BlockSpec((NI // 16, B), lambda i: (i, 0)),
        out_shape=jax.ShapeDtypeStruct((NI, B), jnp.bfloat16),
    )(x)


def _prep_w_body(w_ref, sw_ref):
    w = w_ref[...]
    m = jnp.max(w, axis=-1, keepdims=True)
    ew = jnp.exp(w - m)
    sw_ref[...] = ew / jnp.sum(ew, axis=-1, keepdims=True)


def _prep_w(weights):
    return pl.pallas_call(
        _prep_w_body,
        out_shape=jax.ShapeDtypeStruct((NN, FI), jnp.float32),
    )(weights)


def _log_body(s_ref, o_ref):
    o_ref[...] = jnp.transpose(jnp.log(s_ref[...]))


def _log_kernel(s):
    return pl.pallas_call(
        _log_body,
        grid=(32,),
        in_specs=[pl.BlockSpec((NN // 32, B), lambda i: (i, 0))],
        out_specs=pl.BlockSpec((B, NN // 32), lambda i: (0, i)),
        out_shape=jax.ShapeDtypeStruct((B, NN), jnp.float32),
    )(s)


_sc_mesh = plsc.VectorSubcoreMesh(
    core_axis_name="c", subcore_axis_name="s", num_cores=NC, num_subcores=NS
)


@functools.partial(
    pl.kernel,
    out_type=jax.ShapeDtypeStruct((NN, B), jnp.float32),
    mesh=_sc_mesh,
    compiler_params=pltpu.CompilerParams(needs_layout_passes=False, use_tc_tiling_on_sc=False),
    scratch_types=[
        pltpu.VMEM((NCH, ROWS), jnp.int32),
        pltpu.VMEM((NPT, FI), jnp.float32),
        pltpu.VMEM((ROWS, B // 2), jnp.int32),
        pltpu.VMEM((ROWS, B // 2), jnp.int32),
        pltpu.VMEM((ROWS, B // 2), jnp.int32),
        pltpu.VMEM((ROWS, B // 2), jnp.int32),
        pltpu.VMEM((GRP, B), jnp.float32),
        pltpu.VMEM((GRP, B), jnp.float32),
        pltpu.SemaphoreType.DMA,
        pltpu.SemaphoreType.DMA,
        pltpu.SemaphoreType.DMA,
        pltpu.SemaphoreType.DMA,
        pltpu.SemaphoreType.DMA,
        pltpu.SemaphoreType.DMA,
    ],
)
def _sc_gather_reduce(e_hbm, idx_hbm, w_hbm, out_hbm,
                      idx_v, w_v, rows0, rows1, rows2, rows3, acc0, acc1,
                      semg0, semg1, semg2, semg3, semo0, semo1):
    wid = lax.axis_index("s") * NC + lax.axis_index("c")
    node0 = wid * NPT
    pltpu.sync_copy(idx_hbm.at[wid], idx_v)
    pltpu.sync_copy(w_hbm.at[wid], w_v)

    rows = (rows0, rows1, rows2, rows3)
    accs = (acc0, acc1)
    semg = (semg0, semg1, semg2, semg3)
    semo = (semo0, semo1)

    def compute_chunk(j, rows_ref, acc_ref, part):
        def node_body(n, carry):
            w_vec = w_v[j * CH + n, :]
            acc = [jnp.zeros((LANES,), jnp.float32) for _ in range(2 * NU)]
            for k in range(FI):
                wb = jnp.take(w_vec, jnp.full((LANES,), k, jnp.int32), axis=0)
                r = n * FI + k
                for u in range(NU):
                    w32 = rows_ref[r, pl.ds(u * LANES, LANES)]
                    lo = plsc.bitcast(w32 << 16, jnp.float32)
                    hi = plsc.bitcast(w32 & jnp.int32(-65536), jnp.float32)
                    acc[2 * u] = acc[2 * u] + wb * lo
                    acc[2 * u + 1] = acc[2 * u + 1] + wb * hi
            row = jnp.full((LANES,), part * CH, jnp.int32) + n
            two_iota = 2 * lax.iota(jnp.int32, LANES)
            for u in range(NU):
                plsc.store_scatter(acc_ref, [row, two_iota + (u * 2 * LANES)],
                                   acc[2 * u])
                plsc.store_scatter(acc_ref, [row, two_iota + (u * 2 * LANES + 1)],
                                   acc[2 * u + 1])
            return carry
        lax.fori_loop(0, CH, node_body, 0)

    pltpu.async_copy(e_hbm.at[idx_v.at[0]], rows0, semg0)
    pltpu.async_copy(e_hbm.at[idx_v.at[1]], rows1, semg1)
    pltpu.async_copy(e_hbm.at[idx_v.at[2]], rows2, semg2)
    pltpu.async_copy(e_hbm.at[idx_v.at[3]], rows3, semg3)

    def super_body(i, carry):
        for p in range(8):
            c = 8 * i + p
            buf = p % 4
            q = p // 4
            g = 2 * i + q
            if p % 4 == 0:
                @pl.when(i > 0)
                def _():
                    pltpu.make_async_copy(
                        accs[q],
                        out_hbm.at[pl.ds(node0 + g * GRP, GRP)],
                        semo[q]).wait()
            pltpu.make_async_copy(
                e_hbm.at[idx_v.at[c]], rows[buf], semg[buf]).wait()
            compute_chunk(c, rows[buf], accs[q], p % 4)

            @pl.when(c + 4 < NCH)
            def _():
                pltpu.async_copy(
                    e_hbm.at[idx_v.at[c + 4]], rows[buf], semg[buf])

            if p % 4 == 3:
                pltpu.async_copy(
                    accs[q],
                    out_hbm.at[pl.ds(node0 + g * GRP, GRP)],
                    semo[q])
        return carry

    lax.fori_loop(0, NCH // 8, super_body, 0)
    for q in range(2):
        g = 2 * (NCH // 8 - 1) + q
        pltpu.make_async_copy(
            accs[q], out_hbm.at[pl.ds(node0 + g * GRP, GRP)], semo[q]).wait()


def kernel(x, idxs, weights):
    et = _prep_e(x)
    et32 = lax.bitcast_convert_type(
        et.reshape(NI, B // 2, 2), jnp.int32)
    sw = _prep_w(weights)
    idx32 = idxs.astype(jnp.int32).reshape(NW, NCH, ROWS)
    swr = sw.reshape(NW, NPT, FI)
    s = _sc_gather_reduce(et32, idx32, swr)
    return _log_kernel(s)


# speedup vs baseline: 1.1962x; 1.1962x over previous
import functools

import jax
import jax.numpy as jnp
from jax import lax
from jax.experimental import pallas as pl
from jax.experimental.pallas import tpu as pltpu
from jax.experimental.pallas import tpu_sc as plsc

B = 128
NI = 8192
NN = 16384
FI = 16
NC = 2
NS = 16
NW = NC * NS
NPT = NN // NW
CH = 8
NCH = NPT // CH
ROWS = CH * FI
GRP = 2 * CH
NGRP = NPT // GRP
LANES = 16
NV = B // LANES
NU = B // (2 * LANES)


def _prep_e_body(x_ref, e_ref):
    et = jnp.transpose(jnp.exp(x_ref[...]))
    u = lax.bitcast_convert_type(et, jnp.uint32)
    r = (u + jnp.uint32(0x7FFF) + ((u >> 16) & jnp.uint32(1))) >> 16
    packed = r[:, : B // 2] | (r[:, B // 2:] << 16)
    e_ref[...] = lax.bitcast_convert_type(packed, jnp.int32)


def _prep_e(x):
    return pl.pallas_call(
        _prep_e_body,
        grid=(16,),
        in_specs=[pl.BlockSpec((B, NI // 16), lambda i: (0, i))],
        out_specs=pl.BlockSpec((NI // 16, B // 2), lambda i: (i, 0)),
        out_shape=jax.ShapeDtypeStruct((NI, B // 2), jnp.int32),
    )(x)


def _log_body(s_ref, o_ref):
    o_ref[...] = jnp.transpose(jnp.log(s_ref[...]))


def _log_kernel(s):
    return pl.pallas_call(
        _log_body,
        grid=(32,),
        in_specs=[pl.BlockSpec((NN // 32, B), lambda i: (i, 0))],
        out_specs=pl.BlockSpec((B, NN // 32), lambda i: (0, i)),
        out_shape=jax.ShapeDtypeStruct((B, NN), jnp.float32),
    )(s)


_sc_mesh = plsc.VectorSubcoreMesh(
    core_axis_name="c", subcore_axis_name="s", num_cores=NC, num_subcores=NS
)


@functools.partial(
    pl.kernel,
    out_type=jax.ShapeDtypeStruct((NN, B), jnp.float32),
    mesh=_sc_mesh,
    compiler_params=pltpu.CompilerParams(needs_layout_passes=False, use_tc_tiling_on_sc=False),
    scratch_types=[
        pltpu.VMEM((NCH, ROWS), jnp.int32),
        pltpu.VMEM((NPT, FI), jnp.float32),
        pltpu.VMEM((ROWS, B // 2), jnp.int32),
        pltpu.VMEM((ROWS, B // 2), jnp.int32),
        pltpu.VMEM((ROWS, B // 2), jnp.int32),
        pltpu.VMEM((ROWS, B // 2), jnp.int32),
        pltpu.VMEM((GRP, B), jnp.float32),
        pltpu.VMEM((GRP, B), jnp.float32),
        pltpu.SemaphoreType.DMA,
        pltpu.SemaphoreType.DMA,
        pltpu.SemaphoreType.DMA,
        pltpu.SemaphoreType.DMA,
        pltpu.SemaphoreType.DMA,
        pltpu.SemaphoreType.DMA,
    ],
)
def _sc_gather_reduce(e_hbm, idx_hbm, w_hbm, out_hbm,
                      idx_v, w_v, rows0, rows1, rows2, rows3, acc0, acc1,
                      semg0, semg1, semg2, semg3, semo0, semo1):
    wid = lax.axis_index("s") * NC + lax.axis_index("c")
    node0 = wid * NPT
    pltpu.sync_copy(idx_hbm.at[wid], idx_v)
    pltpu.sync_copy(w_hbm.at[wid], w_v)

    rows = (rows0, rows1, rows2, rows3)
    accs = (acc0, acc1)
    semg = (semg0, semg1, semg2, semg3)
    semo = (semo0, semo1)

    def compute_chunk(j, rows_ref, acc_ref, part):
        def node_body(n, carry):
            ew = jnp.exp(w_v[j * CH + n, :])
            acc = [jnp.zeros((LANES,), jnp.float32) for _ in range(2 * NU)]
            nrm = jnp.zeros((LANES,), jnp.float32)
            for k in range(FI):
                wb = jnp.take(ew, jnp.full((LANES,), k, jnp.int32), axis=0)
                nrm = nrm + wb
                r = n * FI + k
                for u in range(NU):
                    w32 = rows_ref[r, pl.ds(u * LANES, LANES)]
                    lo = plsc.bitcast(w32 << 16, jnp.float32)
                    hi = plsc.bitcast(w32 & jnp.int32(-65536), jnp.float32)
                    acc[u] = acc[u] + wb * lo
                    acc[NU + u] = acc[NU + u] + wb * hi
            rinv = 1.0 / nrm
            row = part * CH + n
            for u in range(2 * NU):
                acc_ref[row, pl.ds(u * LANES, LANES)] = acc[u] * rinv
            return carry
        lax.fori_loop(0, CH, node_body, 0)

    pltpu.async_copy(e_hbm.at[idx_v.at[0]], rows0, semg0)
    pltpu.async_copy(e_hbm.at[idx_v.at[1]], rows1, semg1)
    pltpu.async_copy(e_hbm.at[idx_v.at[2]], rows2, semg2)
    pltpu.async_copy(e_hbm.at[idx_v.at[3]], rows3, semg3)

    def quad_body(i, carry):
        for p in range(4):
            c = 4 * i + p
            buf = p
            q = p // 2
            g = 2 * i + q
            if p % 2 == 0:
                @pl.when(i > 0)
                def _():
                    pltpu.make_async_copy(
                        accs[q],
                        out_hbm.at[pl.ds(node0 + g * GRP, GRP)],
                        semo[q]).wait()
            pltpu.make_async_copy(
                e_hbm.at[idx_v.at[c]], rows[buf], semg[buf]).wait()
            compute_chunk(c, rows[buf], accs[q], p % 2)

            @pl.when(c + 4 < NCH)
            def _():
                pltpu.async_copy(
                    e_hbm.at[idx_v.at[c + 4]], rows[buf], semg[buf])

            if p % 2 == 1:
                pltpu.async_copy(
                    accs[q],
                    out_hbm.at[pl.ds(node0 + g * GRP, GRP)],
                    semo[q])
        return carry

    lax.fori_loop(0, NCH // 4, quad_body, 0)
    for q in range(2):
        g = 2 * (NCH // 4 - 1) + q
        pltpu.make_async_copy(
            accs[q], out_hbm.at[pl.ds(node0 + g * GRP, GRP)],
            semo[q]).wait()


def kernel(x, idxs, weights):
    et = _prep_e(x)
    idx32 = idxs.astype(jnp.int32).reshape(NW, NCH, ROWS)
    wr = weights.reshape(NW, NPT, FI)
    s = _sc_gather_reduce(et, idx32, wr)
    return _log_kernel(s)
